# baseline (speedup 1.0000x reference)
import functools

import jax
import jax.numpy as jnp
from jax import lax
from jax.experimental import pallas as pl
from jax.experimental.pallas import tpu as pltpu
from jax.experimental.pallas import tpu_sc as plsc

_N_BINS = 15
_C = 1000
_N = 16384
_NW = 32
_SPW = _N // _NW
_CH = 16
_NCHUNK = _SPW // _CH


def _sc_body(x_hbm, t_hbm, outc_hbm, outa_hbm,
             xb0, xb1, xb2, tb0, tb1, tb2, oc, oa, dsem):
    wid = lax.axis_index("s") * 2 + lax.axis_index("c")
    wbase = wid * _SPW
    xbufs = (xb0, xb1, xb2)
    tbufs = (tb0, tb1, tb2)

    for h in range(3):
        pltpu.sync_copy(
            t_hbm.at[pl.ds(h + 1, 1), pl.ds(wbase, _SPW)], tbufs[h])

    def _chunk_copies(c, parity):
        return [pltpu.make_async_copy(
            x_hbm.at[pl.ds(wbase + c * _CH, _CH), pl.ds(h, 1)],
            xbufs[h].at[parity], dsem.at[parity]) for h in range(3)]

    for cp in _chunk_copies(0, 0):
        cp.start()

    iota = lax.iota(jnp.int32, 16)
    zi = jnp.zeros((16,), jnp.int32)
    zf = jnp.zeros((16,), jnp.float32)

    def chunk_body(c, carry):
        parity = lax.rem(c, 2)

        @pl.when(c + 1 < _NCHUNK)
        def _prefetch():
            for cp in _chunk_copies(c + 1, 1 - parity):
                cp.start()

        for cp in _chunk_copies(c, parity):
            cp.wait()

        cv = jnp.ones((16,), jnp.float32)
        av = zf
        for h in range(3):
            ref = xbufs[h].at[parity]

            def pbody(i, st):
                vm, vidx, se, vpos = st
                g = plsc.load_gather(ref, [iota, zi, vpos])
                sel = g > vm
                vm = jnp.maximum(vm, g)
                vidx = jnp.where(sel, vpos, vidx)
                se = se + jnp.exp(g)
                return (vm, vidx, se, vpos + 1)

            vm, vidx, se, _ = lax.fori_loop(
                0, _C, pbody,
                (jnp.full((16,), -jnp.inf, jnp.float32), zi, zf, zi),
                unroll=8)
            cv = cv * (jnp.exp(vm) / se)
            tvh = tbufs[h][0, pl.ds(c * _CH, _CH)]
            av = av + (vidx == tvh).astype(jnp.float32)
        oc[pl.ds(c * _CH, _CH)] = cv
        oa[pl.ds(c * _CH, _CH)] = av
        return carry

    lax.fori_loop(0, _NCHUNK, chunk_body, 0, unroll=False)

    pltpu.sync_copy(oc, outc_hbm.at[0, pl.ds(wbase, _SPW)])
    pltpu.sync_copy(oa, outa_hbm.at[0, pl.ds(wbase, _SPW)])


def _bin_body(c_ref, a_ref, out_ref, *, n_total):
    conf = c_ref[...]
    acc = a_ref[...]
    k = jax.lax.broadcasted_iota(jnp.int32, (1, 16), 1)
    kf = k.astype(jnp.float32)
    lows = kf / _N_BINS
    highs = (kf + 1.0) / _N_BINS
    ece = jnp.zeros((1, 1), jnp.float32)
    for i in range(_N_BINS):
        lo = lows[0, i]
        hi = highs[0, i]
        mask = (conf > lo) & (conf <= hi)
        cnt = jnp.sum(mask.astype(jnp.float32))
        cs = jnp.sum(jnp.where(mask, conf, 0.0))
        as_ = jnp.sum(jnp.where(mask, acc, 0.0))
        safe = jnp.maximum(cnt, 1.0)
        term = jnp.abs(cs / safe - as_ / (safe * 3.0)) * (cnt / n_total)
        term = jnp.where(cnt > 0.0, term, 0.0)
        ece = ece + term * jnp.ones((1, 1), jnp.float32)
    out_ref[...] = ece


def kernel(logits, targets):
    n, hds, c = logits.shape
    assert n == _N and hds == 4 and c == _C
    t32 = targets.astype(jnp.int32).T

    mesh = plsc.VectorSubcoreMesh(core_axis_name="c", subcore_axis_name="s")
    sc_fn = functools.partial(
        pl.kernel,
        mesh=mesh,
        compiler_params=pltpu.CompilerParams(needs_layout_passes=False),
        out_type=(jax.ShapeDtypeStruct((1, _N), jnp.float32),
                  jax.ShapeDtypeStruct((1, _N), jnp.float32)),
        scratch_types=[pltpu.VMEM((2, _CH, 1, _C), jnp.float32)
                       for _ in range(3)]
        + [pltpu.VMEM((1, _SPW), jnp.int32) for _ in range(3)]
        + [pltpu.VMEM((_SPW,), jnp.float32) for _ in range(2)]
        + [pltpu.SemaphoreType.DMA((2,))],
    )(_sc_body)
    conf_v, acc_v = sc_fn(logits, t32)

    out = pl.pallas_call(
        functools.partial(_bin_body, n_total=float(n)),
        in_specs=[pl.BlockSpec((1, _N), lambda: (0, 0)),
                  pl.BlockSpec((1, _N), lambda: (0, 0))],
        out_specs=pl.BlockSpec((1, 1), lambda: (0, 0)),
        out_shape=jax.ShapeDtypeStruct((1, 1), jnp.float32),
    )(conf_v, acc_v)
    return out.reshape(1)
